# baseline (speedup 1.0000x reference)
import functools

import jax
import jax.numpy as jnp
from jax import lax
from jax.experimental import pallas as pl
from jax.experimental.pallas import tpu as pltpu
from jax.experimental.pallas import tpu_sc as plsc

NC = 2
NS = 16
NW = NC * NS
LK = 16
EK = 128
ZR = 784
NB = 1024
NODE_Q = 50176
EDGE_Q = NW * EK * 8


def _pad_nodes(n):
    return ((n + NODE_Q - 1) // NODE_Q) * NODE_Q


def _pad_edges(e):
    return ((e + EDGE_Q - 1) // EDGE_Q) * EDGE_Q


def _sc_prop(n_pad, e_pad, n_chunks):
    per_w = e_pad // NW
    iters = per_w // EK
    rps = n_pad // NS
    nz = rps // ZR
    nj = iters // 8
    mesh = plsc.VectorSubcoreMesh(core_axis_name="c", subcore_axis_name="s")

    def body(zt, src, dst, out,
             is0, is1, is2, is3, is4, is5, is6, is7,
             id0, id1, id2, id3, id4, id5, id6, id7,
             r0, r1, r2, r3, zbuf, acc,
             sg0, sg1, sg2, sg3, ss0, ss1, ss2, ss3,
             si0, si1, si2, si3, si4, si5, si6, si7):
        cid = lax.axis_index("c")
        sid = lax.axis_index("s")
        wid = sid * NC + cid
        isb = [is0, is1, is2, is3, is4, is5, is6, is7]
        idb = [id0, id1, id2, id3, id4, id5, id6, id7]
        rwb = [r0, r1, r2, r3]
        sgb = [sg0, sg1, sg2, sg3]
        ssb = [ss0, ss1, ss2, ss3]
        sib = [si0, si1, si2, si3, si4, si5, si6, si7]

        def zi(i, _):
            zbuf[i] = jnp.zeros((LK,), jnp.float32)
            return 0
        lax.fori_loop(0, ZR, zi, 0)

        ebase = wid * per_w
        for c in range(n_chunks):
            for z in range(nz):
                pltpu.sync_copy(zbuf, acc.at[pl.ds(sid * rps + z * ZR, ZR)])
            plsc.subcore_barrier()

            def idx_start(i, b):
                pltpu.async_copy(src.at[pl.ds(ebase + i * EK, EK)],
                                 isb[b], sib[b])
                pltpu.async_copy(dst.at[pl.ds(ebase + i * EK, EK)],
                                 idb[b], sib[b])

            def idx_wait(i, b):
                pltpu.make_async_copy(src.at[pl.ds(ebase + i * EK, EK)],
                                      isb[b], sib[b]).wait()
                pltpu.make_async_copy(dst.at[pl.ds(ebase + i * EK, EK)],
                                      idb[b], sib[b]).wait()

            def gather_start(b8, q):
                pltpu.async_copy(zt.at[c].at[isb[b8]], rwb[q], sgb[q])

            def gather_wait(b8, q):
                pltpu.make_async_copy(zt.at[c].at[isb[b8]],
                                      rwb[q], sgb[q]).wait()

            def scatter_start(b8, q):
                pltpu.async_copy(rwb[q], acc.at[idb[b8]], ssb[q], add=True)

            def scatter_wait(b8, q):
                pltpu.make_async_copy(rwb[q], acc.at[idb[b8]],
                                      ssb[q]).wait()

            for i in range(4):
                idx_start(i, i)

            def eb(j, _):
                for b in range(8):
                    i0 = 8 * j + b
                    q = b % 4
                    idx_wait(i0, b)
                    if b >= 4:
                        scatter_wait(b - 4, q)
                    else:
                        @pl.when(j > 0)
                        def _(b=b, q=q):
                            scatter_wait((b + 4) % 8, q)
                    gather_start(b, q)
                    q2 = (b - 2) % 4
                    b2 = (b - 2) % 8
                    if b >= 2:
                        gather_wait(b2, q2)
                        scatter_start(b2, q2)
                    else:
                        @pl.when(j > 0)
                        def _(b2=b2, q2=q2):
                            gather_wait(b2, q2)
                            scatter_start(b2, q2)
                    b4 = (b + 4) % 8
                    if b < 4:
                        idx_start(i0 + 4, b4)
                    else:
                        @pl.when(j < nj - 1)
                        def _(i0=i0, b4=b4):
                            idx_start(i0 + 4, b4)
                return 0
            lax.fori_loop(0, nj, eb, 0)
            gather_wait(6, 2)
            scatter_start(6, 2)
            gather_wait(7, 3)
            scatter_start(7, 3)
            scatter_wait(4, 0)
            scatter_wait(5, 1)
            scatter_wait(6, 2)
            scatter_wait(7, 3)
            plsc.subcore_barrier()
            pltpu.sync_copy(acc.at[pl.ds(sid * rps, rps)],
                            out.at[cid, c, pl.ds(sid * rps, rps)])

    return pl.kernel(
        body,
        out_type=jax.ShapeDtypeStruct((NC, n_chunks, n_pad, LK), jnp.float32),
        mesh=mesh,
        compiler_params=pltpu.CompilerParams(use_tc_tiling_on_sc=False),
        scratch_types=(
            [pltpu.VMEM((EK,), jnp.int32)] * 16
            + [pltpu.VMEM((EK, LK), jnp.float32)] * 4
            + [pltpu.VMEM((ZR, LK), jnp.float32),
               pltpu.VMEM_SHARED((n_pad, LK), jnp.float32)]
            + [pltpu.SemaphoreType.DMA] * 16
        ),
    )


def _sc_degree(n_pad, e_pad):
    per_w = e_pad // NW
    iters = per_w // EK
    rps = n_pad // NS
    nz = rps // ZR
    mesh = plsc.VectorSubcoreMesh(core_axis_name="c", subcore_axis_name="s")

    half = iters // 2

    def body(dst, out, ones_b, idx_d0, idx_d1, zbuf, acc, si0, si1):
        cid = lax.axis_index("c")
        sid = lax.axis_index("s")
        wid = sid * NC + cid

        def oi(i, _):
            ones_b[i] = jnp.ones((LK,), jnp.float32)
            return 0
        lax.fori_loop(0, EK, oi, 0)

        def zi(i, _):
            zbuf[i] = jnp.zeros((LK,), jnp.float32)
            return 0
        lax.fori_loop(0, ZR, zi, 0)
        for z in range(nz):
            pltpu.sync_copy(zbuf, acc.at[pl.ds(sid * rps + z * ZR, ZR)])
        plsc.subcore_barrier()

        ebase = wid * per_w
        pltpu.async_copy(dst.at[pl.ds(ebase, EK)], idx_d0, si0)
        pltpu.async_copy(dst.at[pl.ds(ebase + EK, EK)], idx_d1, si1)

        def eb(j, _):
            b0 = ebase + (2 * j) * EK
            pltpu.make_async_copy(dst.at[pl.ds(b0, EK)], idx_d0, si0).wait()
            pltpu.sync_copy(ones_b, acc.at[idx_d0], add=True)

            @pl.when(j < half - 1)
            def _():
                pltpu.async_copy(dst.at[pl.ds(b0 + 2 * EK, EK)], idx_d0, si0)
            pltpu.make_async_copy(dst.at[pl.ds(b0 + EK, EK)],
                                  idx_d1, si1).wait()
            pltpu.sync_copy(ones_b, acc.at[idx_d1], add=True)

            @pl.when(j < half - 1)
            def _():
                pltpu.async_copy(dst.at[pl.ds(b0 + 3 * EK, EK)], idx_d1, si1)
            return 0
        lax.fori_loop(0, half, eb, 0)
        plsc.subcore_barrier()
        pltpu.sync_copy(acc.at[pl.ds(sid * rps, rps)],
                        out.at[cid, pl.ds(sid * rps, rps)])

    return pl.kernel(
        body,
        out_type=jax.ShapeDtypeStruct((NC, n_pad, LK), jnp.float32),
        mesh=mesh,
        compiler_params=pltpu.CompilerParams(use_tc_tiling_on_sc=False),
        scratch_types=[
            pltpu.VMEM((EK, LK), jnp.float32),
            pltpu.VMEM((EK,), jnp.int32),
            pltpu.VMEM((EK,), jnp.int32),
            pltpu.VMEM((ZR, LK), jnp.float32),
            pltpu.VMEM_SHARED((n_pad, LK), jnp.float32),
            pltpu.SemaphoreType.DMA,
            pltpu.SemaphoreType.DMA,
        ],
    )


def _tc_prep(n_pad, cx):
    nb = n_pad // NB

    def body(dacc, xp, dinv, xt):
        di = lax.rsqrt(dacc[0] + dacc[1] + 1.0)
        dinv[...] = di
        for c in range(cx):
            xt[c] = di * xp[:, LK * c:LK * (c + 1)]

    return pl.pallas_call(
        body,
        grid=(nb,),
        in_specs=[
            pl.BlockSpec((NC, NB, LK), lambda i: (0, i, 0)),
            pl.BlockSpec((NB, LK * cx), lambda i: (i, 0)),
        ],
        out_specs=[
            pl.BlockSpec((NB, LK), lambda i: (i, 0)),
            pl.BlockSpec((cx, NB, LK), lambda i: (0, i, 0)),
        ],
        out_shape=[
            jax.ShapeDtypeStruct((n_pad, LK), jnp.float32),
            jax.ShapeDtypeStruct((cx, n_pad, LK), jnp.float32),
        ],
    )


def _tc_mm(n_pad, c_in, c_out, relu_pre, two_mats):
    nb = n_pad // NB

    def body(acc, zt, dinv, b_in, W, bm, W2, out):
        cols = []
        for c in range(c_in):
            cols.append(dinv[...] * (acc[0, c] + acc[1, c] + zt[c]))
        h = jnp.concatenate(cols, axis=1) + b_in[...]
        if relu_pre:
            h = jnp.maximum(h, 0.0)
        if two_mats:
            m = jnp.maximum(jnp.dot(h, W[...],
                                    preferred_element_type=jnp.float32)
                            + bm[...], 0.0)
            z = jnp.dot(m, W2[...], preferred_element_type=jnp.float32)
        else:
            z = jnp.dot(h, W[...], preferred_element_type=jnp.float32)
        for c in range(c_out):
            out[c] = dinv[...] * z[:, LK * c:LK * (c + 1)]

    def make(W, bm, W2):
        fi = W.shape[0]
        fm = W.shape[1]
        fo = W2.shape[1] if two_mats else W.shape[1]
        return pl.pallas_call(
            body,
            grid=(nb,),
            in_specs=[
                pl.BlockSpec((NC, c_in, NB, LK), lambda i: (0, 0, i, 0)),
                pl.BlockSpec((c_in, NB, LK), lambda i: (0, i, 0)),
                pl.BlockSpec((NB, LK), lambda i: (i, 0)),
                pl.BlockSpec((1, LK * c_in), lambda i: (0, 0)),
                pl.BlockSpec((fi, fm), lambda i: (0, 0)),
                pl.BlockSpec((1, fm), lambda i: (0, 0)),
                pl.BlockSpec(W2.shape, lambda i: (0, 0)),
            ],
            out_specs=pl.BlockSpec((c_out, NB, LK), lambda i: (0, i, 0)),
            out_shape=jax.ShapeDtypeStruct((c_out, n_pad, LK), jnp.float32),
        )

    return make


def _tc_pool(n_pad):
    nb = n_pad // NB

    def body(acc, zt, dinv, b, bt, sums, cnt):
        i = pl.program_id(0)
        x3 = dinv[...] * (acc[0, 0] + acc[1, 0] + zt[0]) + b[...]
        gid = lax.broadcasted_iota(jnp.int32, (128, NB), 0)
        m = (jnp.broadcast_to(bt[0], (128, NB)) == gid).astype(jnp.float32)
        s = jnp.dot(m, x3, preferred_element_type=jnp.float32)
        c = jnp.dot(m, jnp.ones_like(x3), preferred_element_type=jnp.float32)

        @pl.when(i == 0)
        def _():
            sums[...] = s
            cnt[...] = c

        @pl.when(i > 0)
        def _():
            sums[...] += s
            cnt[...] += c

    return pl.pallas_call(
        body,
        grid=(nb,),
        in_specs=[
            pl.BlockSpec((NC, 1, NB, LK), lambda i: (0, 0, i, 0)),
            pl.BlockSpec((1, NB, LK), lambda i: (0, i, 0)),
            pl.BlockSpec((NB, LK), lambda i: (i, 0)),
            pl.BlockSpec((1, LK), lambda i: (0, 0)),
            pl.BlockSpec((1, 1, NB), lambda i: (i, 0, 0)),
        ],
        out_specs=[
            pl.BlockSpec((128, LK), lambda i: (0, 0)),
            pl.BlockSpec((128, LK), lambda i: (0, 0)),
        ],
        out_shape=[
            jax.ShapeDtypeStruct((128, LK), jnp.float32),
            jax.ShapeDtypeStruct((128, LK), jnp.float32),
        ],
    )


def _tc_final():
    def body(so, co, sl, cl, Ao, Al, bf, out):
        xo = so[...] / jnp.maximum(co[...], 1.0)
        xl = sl[...] / jnp.maximum(cl[...], 1.0)
        logits = (jnp.dot(xo, Ao[...], preferred_element_type=jnp.float32)
                  + jnp.dot(xl, Al[...], preferred_element_type=jnp.float32)
                  + bf[...])
        col = lax.broadcasted_iota(jnp.int32, (128, 128), 1)
        valid = col < 2
        neg = jnp.float32(-1e30)
        lm = jnp.where(valid, logits, neg)
        mx = jnp.max(lm, axis=1, keepdims=True)
        se = jnp.sum(jnp.where(valid, jnp.exp(logits - mx), 0.0),
                     axis=1, keepdims=True)
        out[...] = logits - mx - jnp.log(se)

    return pl.pallas_call(
        body,
        out_shape=jax.ShapeDtypeStruct((128, 128), jnp.float32),
    )


def _branch(x, edge_index, batch, n_pad, c_feat):
    n = x.shape[0]
    e = edge_index.shape[1]
    e_pad = _pad_edges(e)
    xp = jnp.pad(x, ((0, n_pad - n), (0, LK * c_feat - x.shape[1])))
    src = jnp.pad(edge_index[0], (0, e_pad - e), constant_values=n_pad - 1)
    dst = jnp.pad(edge_index[1], (0, e_pad - e), constant_values=n_pad - 1)
    bt = jnp.pad(batch, (0, n_pad - n), constant_values=200)
    bt3 = bt.reshape(n_pad // NB, 1, NB)
    return xp, src, dst, bt3, e_pad


def kernel(x_origin, edge_index_origin, batch_origin, x_line,
           edge_index_line, batch_line, W1, b1, W2, b2, W5, b5,
           W3, b3, W4, b4, Wfc, bfc):
    f32 = jnp.float32
    n_o = x_origin.shape[0]
    n_l = x_line.shape[0]
    np_o = _pad_nodes(n_o)
    np_l = _pad_nodes(n_l)

    xo_p, src_o, dst_o, bt3_o, ep_o = _branch(
        x_origin, edge_index_origin, batch_origin, np_o, 2)
    xl_p, src_l, dst_l, bt3_l, ep_l = _branch(
        x_line, edge_index_line, batch_line, np_l, 4)

    W1p = jnp.pad(W1, ((0, 32 - W1.shape[0]), (0, 0)))
    W5p = jnp.pad(W5, ((0, 0), (0, LK - W5.shape[1])))
    W3p = jnp.pad(W3, ((0, 64 - W3.shape[0]), (0, 0)))
    W4p = jnp.pad(W4, ((0, 0), (0, LK - W4.shape[1])))
    b1r = b1.reshape(1, -1)
    b2r = b2.reshape(1, -1)
    b3r = b3.reshape(1, -1)
    b5p = jnp.pad(b5, (0, LK - b5.shape[0])).reshape(1, LK)
    b4p = jnp.pad(b4, (0, LK - b4.shape[0])).reshape(1, LK)
    zero16 = jnp.zeros((1, LK), f32)
    zero32 = jnp.zeros((1, 32), f32)
    zero64 = jnp.zeros((1, 64), f32)
    eye16 = jnp.eye(LK, dtype=f32)
    Ao = jnp.zeros((LK, 128), f32).at[:5, :2].set(Wfc[:5])
    Al = jnp.zeros((LK, 128), f32).at[:5, :2].set(Wfc[5:])
    bfp = jnp.zeros((1, 128), f32).at[0, :2].set(bfc)

    dacc_o = _sc_degree(np_o, ep_o)(dst_o)
    dinv_o, xt_o = _tc_prep(np_o, 2)(dacc_o, xo_p)
    a1 = _sc_prop(np_o, ep_o, 2)(xt_o, src_o, dst_o)
    z2t = _tc_mm(np_o, 2, 4, False, True)(W1p, b1r, W2)(
        a1, xt_o, dinv_o, zero32, W1p, b1r, W2)
    a2 = _sc_prop(np_o, ep_o, 4)(z2t, src_o, dst_o)
    z3t = _tc_mm(np_o, 4, 1, True, False)(W5p, zero16, eye16)(
        a2, z2t, dinv_o, b2r, W5p, zero16, eye16)
    a3 = _sc_prop(np_o, ep_o, 1)(z3t, src_o, dst_o)
    sums_o, cnt_o = _tc_pool(np_o)(a3, z3t, dinv_o, b5p, bt3_o)

    dacc_l = _sc_degree(np_l, ep_l)(dst_l)
    dinv_l, xt_l = _tc_prep(np_l, 4)(dacc_l, xl_p)
    c1 = _sc_prop(np_l, ep_l, 4)(xt_l, src_l, dst_l)
    z4t = _tc_mm(np_l, 4, 1, False, True)(W3p, b3r, W4p)(
        c1, xt_l, dinv_l, zero64, W3p, b3r, W4p)
    c2 = _sc_prop(np_l, ep_l, 1)(z4t, src_l, dst_l)
    sums_l, cnt_l = _tc_pool(np_l)(c2, z4t, dinv_l, b4p, bt3_l)

    out = _tc_final()(sums_o, cnt_o, sums_l, cnt_l, Ao, Al, bfp)
    return out[:, :2]


# speedup vs baseline: 1.0767x; 1.0767x over previous
import functools

import jax
import jax.numpy as jnp
from jax import lax
from jax.experimental import pallas as pl
from jax.experimental.pallas import tpu as pltpu
from jax.experimental.pallas import tpu_sc as plsc

NC = 2
NS = 16
NW = NC * NS
LK = 16
EK = 128
ZR = 784
NB = 1024
NODE_Q = 50176
EDGE_Q = NW * EK * 4


def _pad_nodes(n):
    return ((n + NODE_Q - 1) // NODE_Q) * NODE_Q


def _pad_edges(e):
    return ((e + EDGE_Q - 1) // EDGE_Q) * EDGE_Q


def _sc_prop(n_pad, e_pad, n_chunks):
    per_w = e_pad // NW
    iters = per_w // EK
    rps = n_pad // NS
    nz = rps // ZR
    nj = iters // 4
    mesh = plsc.VectorSubcoreMesh(core_axis_name="c", subcore_axis_name="s")

    def body(zt, src, dst, out,
             is0, is1, is2, is3, id0, id1, id2, id3,
             r0, r1, r2, r3, zbuf, acc,
             sg0, sg1, sg2, sg3, si0, si1, si2, si3):
        cid = lax.axis_index("c")
        sid = lax.axis_index("s")
        wid = sid * NC + cid
        isb = [is0, is1, is2, is3]
        idb = [id0, id1, id2, id3]
        rwb = [r0, r1, r2, r3]
        sgb = [sg0, sg1, sg2, sg3]
        sib = [si0, si1, si2, si3]

        def zi(i, _):
            zbuf[i] = jnp.zeros((LK,), jnp.float32)
            return 0
        lax.fori_loop(0, ZR, zi, 0)

        ebase = wid * per_w
        for c in range(n_chunks):
            for z in range(nz):
                pltpu.sync_copy(zbuf, acc.at[pl.ds(sid * rps + z * ZR, ZR)])
            plsc.subcore_barrier()

            def idx_start(i, b):
                pltpu.async_copy(src.at[pl.ds(ebase + i * EK, EK)],
                                 isb[b], sib[b])
                pltpu.async_copy(dst.at[pl.ds(ebase + i * EK, EK)],
                                 idb[b], sib[b])

            def idx_wait(i, b):
                pltpu.make_async_copy(src.at[pl.ds(ebase + i * EK, EK)],
                                      isb[b], sib[b]).wait()
                pltpu.make_async_copy(dst.at[pl.ds(ebase + i * EK, EK)],
                                      idb[b], sib[b]).wait()

            def gather_start(b):
                pltpu.async_copy(zt.at[c].at[isb[b]], rwb[b], sgb[b])

            def gather_wait_scatter(b):
                pltpu.make_async_copy(zt.at[c].at[isb[b]],
                                      rwb[b], sgb[b]).wait()
                pltpu.sync_copy(rwb[b], acc.at[idb[b]], add=True)

            idx_start(0, 0)
            idx_start(1, 1)

            def eb(j, _):
                i0 = 4 * j
                for b in range(4):
                    i = i0 + b
                    idx_wait(i, b)
                    gather_start(b)
                    if b >= 2:
                        gather_wait_scatter(b - 2)
                    else:
                        @pl.when(j > 0)
                        def _(b=b):
                            gather_wait_scatter(b + 2)
                    if b < 2:
                        idx_start(i + 2, b + 2)
                    else:
                        @pl.when(j < nj - 1)
                        def _(i=i, b=b):
                            idx_start(i + 2, b - 2)
                return 0
            lax.fori_loop(0, nj, eb, 0)
            gather_wait_scatter(2)
            gather_wait_scatter(3)
            plsc.subcore_barrier()
            pltpu.sync_copy(acc.at[pl.ds(sid * rps, rps)],
                            out.at[cid, c, pl.ds(sid * rps, rps)])

    return pl.kernel(
        body,
        out_type=jax.ShapeDtypeStruct((NC, n_chunks, n_pad, LK), jnp.float32),
        mesh=mesh,
        compiler_params=pltpu.CompilerParams(use_tc_tiling_on_sc=False),
        scratch_types=(
            [pltpu.VMEM((EK,), jnp.int32)] * 8
            + [pltpu.VMEM((EK, LK), jnp.float32)] * 4
            + [pltpu.VMEM((ZR, LK), jnp.float32),
               pltpu.VMEM_SHARED((n_pad, LK), jnp.float32)]
            + [pltpu.SemaphoreType.DMA] * 8
        ),
    )


def _sc_degree(n_pad, e_pad):
    per_w = e_pad // NW
    iters = per_w // EK
    rps = n_pad // NS
    nz = rps // ZR
    mesh = plsc.VectorSubcoreMesh(core_axis_name="c", subcore_axis_name="s")

    half = iters // 2

    def body(dst, out, ones_b, idx_d0, idx_d1, zbuf, acc, si0, si1):
        cid = lax.axis_index("c")
        sid = lax.axis_index("s")
        wid = sid * NC + cid

        def oi(i, _):
            ones_b[i] = jnp.ones((LK,), jnp.float32)
            return 0
        lax.fori_loop(0, EK, oi, 0)

        def zi(i, _):
            zbuf[i] = jnp.zeros((LK,), jnp.float32)
            return 0
        lax.fori_loop(0, ZR, zi, 0)
        for z in range(nz):
            pltpu.sync_copy(zbuf, acc.at[pl.ds(sid * rps + z * ZR, ZR)])
        plsc.subcore_barrier()

        ebase = wid * per_w
        pltpu.async_copy(dst.at[pl.ds(ebase, EK)], idx_d0, si0)
        pltpu.async_copy(dst.at[pl.ds(ebase + EK, EK)], idx_d1, si1)

        def eb(j, _):
            b0 = ebase + (2 * j) * EK
            pltpu.make_async_copy(dst.at[pl.ds(b0, EK)], idx_d0, si0).wait()
            pltpu.sync_copy(ones_b, acc.at[idx_d0], add=True)

            @pl.when(j < half - 1)
            def _():
                pltpu.async_copy(dst.at[pl.ds(b0 + 2 * EK, EK)], idx_d0, si0)
            pltpu.make_async_copy(dst.at[pl.ds(b0 + EK, EK)],
                                  idx_d1, si1).wait()
            pltpu.sync_copy(ones_b, acc.at[idx_d1], add=True)

            @pl.when(j < half - 1)
            def _():
                pltpu.async_copy(dst.at[pl.ds(b0 + 3 * EK, EK)], idx_d1, si1)
            return 0
        lax.fori_loop(0, half, eb, 0)
        plsc.subcore_barrier()
        pltpu.sync_copy(acc.at[pl.ds(sid * rps, rps)],
                        out.at[cid, pl.ds(sid * rps, rps)])

    return pl.kernel(
        body,
        out_type=jax.ShapeDtypeStruct((NC, n_pad, LK), jnp.float32),
        mesh=mesh,
        compiler_params=pltpu.CompilerParams(use_tc_tiling_on_sc=False),
        scratch_types=[
            pltpu.VMEM((EK, LK), jnp.float32),
            pltpu.VMEM((EK,), jnp.int32),
            pltpu.VMEM((EK,), jnp.int32),
            pltpu.VMEM((ZR, LK), jnp.float32),
            pltpu.VMEM_SHARED((n_pad, LK), jnp.float32),
            pltpu.SemaphoreType.DMA,
            pltpu.SemaphoreType.DMA,
        ],
    )


def _tc_prep(n_pad, cx):
    nb = n_pad // NB

    def body(dacc, xp, dinv, xt):
        di = lax.rsqrt(dacc[0] + dacc[1] + 1.0)
        dinv[...] = di
        for c in range(cx):
            xt[c] = di * xp[:, LK * c:LK * (c + 1)]

    return pl.pallas_call(
        body,
        grid=(nb,),
        in_specs=[
            pl.BlockSpec((NC, NB, LK), lambda i: (0, i, 0)),
            pl.BlockSpec((NB, LK * cx), lambda i: (i, 0)),
        ],
        out_specs=[
            pl.BlockSpec((NB, LK), lambda i: (i, 0)),
            pl.BlockSpec((cx, NB, LK), lambda i: (0, i, 0)),
        ],
        out_shape=[
            jax.ShapeDtypeStruct((n_pad, LK), jnp.float32),
            jax.ShapeDtypeStruct((cx, n_pad, LK), jnp.float32),
        ],
    )


def _tc_mm(n_pad, c_in, c_out, relu_pre, two_mats):
    nb = n_pad // NB

    def body(acc, zt, dinv, b_in, W, bm, W2, out):
        cols = []
        for c in range(c_in):
            cols.append(dinv[...] * (acc[0, c] + acc[1, c] + zt[c]))
        h = jnp.concatenate(cols, axis=1) + b_in[...]
        if relu_pre:
            h = jnp.maximum(h, 0.0)
        if two_mats:
            m = jnp.maximum(jnp.dot(h, W[...],
                                    preferred_element_type=jnp.float32)
                            + bm[...], 0.0)
            z = jnp.dot(m, W2[...], preferred_element_type=jnp.float32)
        else:
            z = jnp.dot(h, W[...], preferred_element_type=jnp.float32)
        for c in range(c_out):
            out[c] = dinv[...] * z[:, LK * c:LK * (c + 1)]

    def make(W, bm, W2):
        fi = W.shape[0]
        fm = W.shape[1]
        fo = W2.shape[1] if two_mats else W.shape[1]
        return pl.pallas_call(
            body,
            grid=(nb,),
            in_specs=[
                pl.BlockSpec((NC, c_in, NB, LK), lambda i: (0, 0, i, 0)),
                pl.BlockSpec((c_in, NB, LK), lambda i: (0, i, 0)),
                pl.BlockSpec((NB, LK), lambda i: (i, 0)),
                pl.BlockSpec((1, LK * c_in), lambda i: (0, 0)),
                pl.BlockSpec((fi, fm), lambda i: (0, 0)),
                pl.BlockSpec((1, fm), lambda i: (0, 0)),
                pl.BlockSpec(W2.shape, lambda i: (0, 0)),
            ],
            out_specs=pl.BlockSpec((c_out, NB, LK), lambda i: (0, i, 0)),
            out_shape=jax.ShapeDtypeStruct((c_out, n_pad, LK), jnp.float32),
        )

    return make


def _tc_pool(n_pad):
    nb = n_pad // NB

    def body(acc, zt, dinv, b, bt, sums, cnt):
        i = pl.program_id(0)
        x3 = dinv[...] * (acc[0, 0] + acc[1, 0] + zt[0]) + b[...]
        gid = lax.broadcasted_iota(jnp.int32, (128, NB), 0)
        m = (jnp.broadcast_to(bt[0], (128, NB)) == gid).astype(jnp.float32)
        s = jnp.dot(m, x3, preferred_element_type=jnp.float32)
        c = jnp.dot(m, jnp.ones_like(x3), preferred_element_type=jnp.float32)

        @pl.when(i == 0)
        def _():
            sums[...] = s
            cnt[...] = c

        @pl.when(i > 0)
        def _():
            sums[...] += s
            cnt[...] += c

    return pl.pallas_call(
        body,
        grid=(nb,),
        in_specs=[
            pl.BlockSpec((NC, 1, NB, LK), lambda i: (0, 0, i, 0)),
            pl.BlockSpec((1, NB, LK), lambda i: (0, i, 0)),
            pl.BlockSpec((NB, LK), lambda i: (i, 0)),
            pl.BlockSpec((1, LK), lambda i: (0, 0)),
            pl.BlockSpec((1, 1, NB), lambda i: (i, 0, 0)),
        ],
        out_specs=[
            pl.BlockSpec((128, LK), lambda i: (0, 0)),
            pl.BlockSpec((128, LK), lambda i: (0, 0)),
        ],
        out_shape=[
            jax.ShapeDtypeStruct((128, LK), jnp.float32),
            jax.ShapeDtypeStruct((128, LK), jnp.float32),
        ],
    )


def _tc_final():
    def body(so, co, sl, cl, Ao, Al, bf, out):
        xo = so[...] / jnp.maximum(co[...], 1.0)
        xl = sl[...] / jnp.maximum(cl[...], 1.0)
        logits = (jnp.dot(xo, Ao[...], preferred_element_type=jnp.float32)
                  + jnp.dot(xl, Al[...], preferred_element_type=jnp.float32)
                  + bf[...])
        col = lax.broadcasted_iota(jnp.int32, (128, 128), 1)
        valid = col < 2
        neg = jnp.float32(-1e30)
        lm = jnp.where(valid, logits, neg)
        mx = jnp.max(lm, axis=1, keepdims=True)
        se = jnp.sum(jnp.where(valid, jnp.exp(logits - mx), 0.0),
                     axis=1, keepdims=True)
        out[...] = logits - mx - jnp.log(se)

    return pl.pallas_call(
        body,
        out_shape=jax.ShapeDtypeStruct((128, 128), jnp.float32),
    )


def _branch(x, edge_index, batch, n_pad, c_feat):
    n = x.shape[0]
    e = edge_index.shape[1]
    e_pad = _pad_edges(e)
    xp = jnp.pad(x, ((0, n_pad - n), (0, LK * c_feat - x.shape[1])))
    src = jnp.pad(edge_index[0], (0, e_pad - e), constant_values=n_pad - 1)
    dst = jnp.pad(edge_index[1], (0, e_pad - e), constant_values=n_pad - 1)
    bt = jnp.pad(batch, (0, n_pad - n), constant_values=200)
    bt3 = bt.reshape(n_pad // NB, 1, NB)
    return xp, src, dst, bt3, e_pad


def kernel(x_origin, edge_index_origin, batch_origin, x_line,
           edge_index_line, batch_line, W1, b1, W2, b2, W5, b5,
           W3, b3, W4, b4, Wfc, bfc):
    f32 = jnp.float32
    n_o = x_origin.shape[0]
    n_l = x_line.shape[0]
    np_o = _pad_nodes(n_o)
    np_l = _pad_nodes(n_l)

    xo_p, src_o, dst_o, bt3_o, ep_o = _branch(
        x_origin, edge_index_origin, batch_origin, np_o, 2)
    xl_p, src_l, dst_l, bt3_l, ep_l = _branch(
        x_line, edge_index_line, batch_line, np_l, 4)

    W1p = jnp.pad(W1, ((0, 32 - W1.shape[0]), (0, 0)))
    W5p = jnp.pad(W5, ((0, 0), (0, LK - W5.shape[1])))
    W3p = jnp.pad(W3, ((0, 64 - W3.shape[0]), (0, 0)))
    W4p = jnp.pad(W4, ((0, 0), (0, LK - W4.shape[1])))
    b1r = b1.reshape(1, -1)
    b2r = b2.reshape(1, -1)
    b3r = b3.reshape(1, -1)
    b5p = jnp.pad(b5, (0, LK - b5.shape[0])).reshape(1, LK)
    b4p = jnp.pad(b4, (0, LK - b4.shape[0])).reshape(1, LK)
    zero16 = jnp.zeros((1, LK), f32)
    zero32 = jnp.zeros((1, 32), f32)
    zero64 = jnp.zeros((1, 64), f32)
    eye16 = jnp.eye(LK, dtype=f32)
    Ao = jnp.zeros((LK, 128), f32).at[:5, :2].set(Wfc[:5])
    Al = jnp.zeros((LK, 128), f32).at[:5, :2].set(Wfc[5:])
    bfp = jnp.zeros((1, 128), f32).at[0, :2].set(bfc)

    dacc_o = _sc_degree(np_o, ep_o)(dst_o)
    dinv_o, xt_o = _tc_prep(np_o, 2)(dacc_o, xo_p)
    a1 = _sc_prop(np_o, ep_o, 2)(xt_o, src_o, dst_o)
    z2t = _tc_mm(np_o, 2, 4, False, True)(W1p, b1r, W2)(
        a1, xt_o, dinv_o, zero32, W1p, b1r, W2)
    a2 = _sc_prop(np_o, ep_o, 4)(z2t, src_o, dst_o)
    z3t = _tc_mm(np_o, 4, 1, True, False)(W5p, zero16, eye16)(
        a2, z2t, dinv_o, b2r, W5p, zero16, eye16)
    a3 = _sc_prop(np_o, ep_o, 1)(z3t, src_o, dst_o)
    sums_o, cnt_o = _tc_pool(np_o)(a3, z3t, dinv_o, b5p, bt3_o)

    dacc_l = _sc_degree(np_l, ep_l)(dst_l)
    dinv_l, xt_l = _tc_prep(np_l, 4)(dacc_l, xl_p)
    c1 = _sc_prop(np_l, ep_l, 4)(xt_l, src_l, dst_l)
    z4t = _tc_mm(np_l, 4, 1, False, True)(W3p, b3r, W4p)(
        c1, xt_l, dinv_l, zero64, W3p, b3r, W4p)
    c2 = _sc_prop(np_l, ep_l, 1)(z4t, src_l, dst_l)
    sums_l, cnt_l = _tc_pool(np_l)(c2, z4t, dinv_l, b4p, bt3_l)

    out = _tc_final()(sums_o, cnt_o, sums_l, cnt_l, Ao, Al, bfp)
    return out[:, :2]
